# baseline (speedup 1.0000x reference)
import functools

import jax
import jax.numpy as jnp
from jax import lax
from jax.experimental import pallas as pl
from jax.experimental.pallas import tpu as pltpu
from jax.experimental.pallas import tpu_sc as plsc

_N = 10000
_E = 320000
_D = 128
_L = 3
_EPS = 1e-5

_NC, _NS = 2, 16
_NW = _NC * _NS
_EPW = _E // _NW
_CH = 128
_NFULL = _EPW // _CH
_TAIL = _EPW - _NFULL * _CH
_NPT = 632
_NPAD = _NS * _NPT

_ET = 1280
_NT = 1000


def _ln(x, g, b):
    m = jnp.mean(x, axis=-1, keepdims=True)
    v = jnp.mean(jnp.square(x - m), axis=-1, keepdims=True)
    return (x - m) * lax.rsqrt(v + _EPS) * g + b


def _pack_bf16(x):
    b = jax.lax.bitcast_convert_type(x.astype(jnp.bfloat16),
                                     jnp.uint16).astype(jnp.int32)
    return b[:, :64] | (b[:, 64:] << 16)


def _unpack_bf16(w):
    lo = jax.lax.bitcast_convert_type(w << 16, jnp.float32)
    hi = jax.lax.bitcast_convert_type(w & jnp.int32(-65536), jnp.float32)
    return jnp.concatenate([lo, hi], axis=1)


def _mm_body(x_ref, w_ref, o_ref):
    o_ref[...] = _pack_bf16(jnp.dot(x_ref[...], w_ref[...],
                                    preferred_element_type=jnp.float32))


def _rows_matmul(x, w):
    n, k = x.shape
    return pl.pallas_call(
        _mm_body,
        grid=(n // _NT,),
        in_specs=[pl.BlockSpec((_NT, k), lambda i: (i, 0)),
                  pl.BlockSpec(w.shape, lambda i: (0, 0))],
        out_specs=pl.BlockSpec((_NT, w.shape[1] // 2), lambda i: (i, 0)),
        out_shape=jax.ShapeDtypeStruct((n, w.shape[1] // 2), jnp.int32),
    )(x, w)


def _edge_body(g_ref, bx_ref, w1_ref, b1_ref, g1_ref, B1_ref,
               w2_ref, b2_ref, g2_ref, B2_ref, o_ref):
    u = (_unpack_bf16(g_ref[...])
         + jnp.dot(bx_ref[...], w1_ref[...], preferred_element_type=jnp.float32)
         + b1_ref[...])
    u = jnp.maximum(_ln(u, g1_ref[...], B1_ref[...]), 0.0)
    t = jnp.dot(u, w2_ref[...], preferred_element_type=jnp.float32) + b2_ref[...]
    o_ref[...] = jnp.maximum(_ln(t, g2_ref[...], B2_ref[...]), 0.0)


def _edge_mlp(gath, bond, w1b, b1, g1, B1, w2, b2, g2, B2):
    row = pl.BlockSpec((_ET, _D), lambda i: (i, 0))
    rowp = pl.BlockSpec((_ET, _D // 2), lambda i: (i, 0))

    def full(a):
        return pl.BlockSpec(a.shape, lambda i: (0,) * a.ndim)

    return pl.pallas_call(
        _edge_body,
        grid=(_E // _ET,),
        in_specs=[rowp, row, full(w1b), full(b1), full(g1), full(B1),
                  full(w2), full(b2), full(g2), full(B2)],
        out_specs=row,
        out_shape=jax.ShapeDtypeStruct((_E, _D), jnp.float32),
    )(gath, bond, w1b, b1, g1, B1, w2, b2, g2, B2)


def _node_body(ax_ref, p0_ref, p1_ref, w1a_ref, w1b_ref, b1_ref, g1_ref,
               B1_ref, w2_ref, b2_ref, g2_ref, B2_ref, wn_ref,
               h_ref, hp_ref):
    agg = p0_ref[...] + p1_ref[...]
    t = (jnp.dot(ax_ref[...], w1a_ref[...], preferred_element_type=jnp.float32)
         + jnp.dot(agg, w1b_ref[...], preferred_element_type=jnp.float32)
         + b1_ref[...])
    t = jnp.maximum(_ln(t, g1_ref[...], B1_ref[...]), 0.0)
    t2 = jnp.dot(t, w2_ref[...], preferred_element_type=jnp.float32) + b2_ref[...]
    h = jnp.maximum(_ln(t2, g2_ref[...], B2_ref[...]), 0.0)
    h_ref[...] = h
    hp_ref[...] = _pack_bf16(jnp.dot(h, wn_ref[...],
                                     preferred_element_type=jnp.float32))


def _node_mlp(atom, part0, part1, w1a, w1b, b1, g1, B1, w2, b2, g2, B2,
              wnext):
    grid = _N // _NT
    row = pl.BlockSpec((_NT, _D), lambda i: (i, 0))
    rowp = pl.BlockSpec((_NT, _D // 2), lambda i: (i, 0))

    def full(a):
        return pl.BlockSpec(a.shape, lambda i: (0,) * a.ndim)

    return pl.pallas_call(
        _node_body,
        grid=(grid,),
        in_specs=[row, row, row, full(w1a), full(w1b), full(b1), full(g1),
                  full(B1), full(w2), full(b2), full(g2), full(B2),
                  full(wnext)],
        out_specs=(row, rowp),
        out_shape=(jax.ShapeDtypeStruct((_N, _D), jnp.float32),
                   jax.ShapeDtypeStruct((_N, _D // 2), jnp.int32)),
    )(atom, part0, part1, w1a, w1b, b1, g1, B1, w2, b2, g2, B2, wnext)


def _sc_gather(hp, src):
    mesh = plsc.VectorSubcoreMesh(core_axis_name="c", subcore_axis_name="s")

    @functools.partial(
        pl.kernel, mesh=mesh,
        out_type=jax.ShapeDtypeStruct((_E, _D // 2), jnp.int32),
        scratch_types=[pltpu.VMEM((_CH,), jnp.int32),
                       pltpu.VMEM((_CH, _D // 2), jnp.int32),
                       pltpu.VMEM((_TAIL,), jnp.int32),
                       pltpu.VMEM((_TAIL, _D // 2), jnp.int32),
                       pltpu.SemaphoreType.DMA],
        compiler_params=pltpu.CompilerParams(use_tc_tiling_on_sc=False),
    )
    def gather_k(hp_hbm, src_hbm, out_hbm, idx_v, rows_v, idxt_v, rowst_v, sem):
        w = lax.axis_index("c") * _NS + lax.axis_index("s")
        base = w * _EPW

        def body(i, carry):
            off = base + i * _CH
            pltpu.sync_copy(src_hbm.at[pl.ds(off, _CH)], idx_v)
            pltpu.async_copy(hp_hbm.at[idx_v], rows_v, sem).wait()
            pltpu.sync_copy(rows_v, out_hbm.at[pl.ds(off, _CH)])
            return carry

        lax.fori_loop(0, _NFULL, body, 0)
        off = base + _NFULL * _CH
        pltpu.sync_copy(src_hbm.at[pl.ds(off, _TAIL)], idxt_v)
        pltpu.async_copy(hp_hbm.at[idxt_v], rowst_v, sem).wait()
        pltpu.sync_copy(rowst_v, out_hbm.at[pl.ds(off, _TAIL)])

    return gather_k(hp, src)


def _sc_scatter(bh, dst, zeros_tile):
    mesh = plsc.VectorSubcoreMesh(core_axis_name="c", subcore_axis_name="s")

    @functools.partial(
        pl.kernel, mesh=mesh,
        out_type=(jax.ShapeDtypeStruct((_NPAD, _D), jnp.float32),
                  jax.ShapeDtypeStruct((_NPAD, _D), jnp.float32)),
        scratch_types=[pltpu.VMEM_SHARED((_NPAD, _D), jnp.float32),
                       pltpu.VMEM((_CH,), jnp.int32),
                       pltpu.VMEM((_CH, _D), jnp.float32),
                       pltpu.VMEM((_TAIL,), jnp.int32),
                       pltpu.VMEM((_TAIL, _D), jnp.float32),
                       pltpu.SemaphoreType.DMA],
    )
    def scatter_k(bh_hbm, dst_hbm, z_hbm, out0_hbm, out1_hbm, acc, idx_v,
                  rows_v, idxt_v, rowst_v, sem):
        c = lax.axis_index("c")
        s = lax.axis_index("s")
        pltpu.sync_copy(z_hbm, acc.at[pl.ds(s * _NPT, _NPT)])
        plsc.subcore_barrier()
        base = (c * _NS + s) * _EPW

        def body(i, carry):
            off = base + i * _CH
            pltpu.sync_copy(dst_hbm.at[pl.ds(off, _CH)], idx_v)
            pltpu.sync_copy(bh_hbm.at[pl.ds(off, _CH)], rows_v)
            pltpu.sync_copy(rows_v, acc.at[idx_v], add=True)
            return carry

        lax.fori_loop(0, _NFULL, body, 0)
        off = base + _NFULL * _CH
        pltpu.sync_copy(dst_hbm.at[pl.ds(off, _TAIL)], idxt_v)
        pltpu.sync_copy(bh_hbm.at[pl.ds(off, _TAIL)], rowst_v)
        pltpu.sync_copy(rowst_v, acc.at[idxt_v], add=True)
        plsc.subcore_barrier()

        @pl.when(c == 0)
        def _():
            pltpu.sync_copy(acc.at[pl.ds(s * _NPT, _NPT)],
                            out0_hbm.at[pl.ds(s * _NPT, _NPT)])

        @pl.when(c == 1)
        def _():
            pltpu.sync_copy(acc.at[pl.ds(s * _NPT, _NPT)],
                            out1_hbm.at[pl.ds(s * _NPT, _NPT)])

    return scatter_k(bh, dst, zeros_tile)


def kernel(atom_x, bond_x, subtree_h, bW1, bb1, bg1, bB1, bW2, bb2, bg2,
           bB2, tW1, tb1, tg1, tB1, tW2, tb2, tg2, tB2, edge_index):
    src = edge_index[0]
    dst = edge_index[1]
    zeros_tile = jnp.zeros((_NPT, _D), jnp.float32)
    bond16 = bond_x.astype(jnp.bfloat16)

    hp = _rows_matmul(subtree_h, bW1[0, :_D, :])
    h = subtree_h
    for l in range(_L):
        g = _sc_gather(hp, src)
        bh = _edge_mlp(g, bond16, bW1[l, _D:, :].astype(jnp.bfloat16),
                       bb1[l][None], bg1[l][None],
                       bB1[l][None], bW2[l], bb2[l][None], bg2[l][None],
                       bB2[l][None])
        part0, part1 = _sc_scatter(bh, dst, zeros_tile)
        wnext = bW1[(l + 1) % _L, :_D, :]
        h, hp = _node_mlp(atom_x, part0, part1, tW1[l, :_D, :], tW1[l, _D:, :],
                          tb1[l][None], tg1[l][None], tB1[l][None], tW2[l],
                          tb2[l][None], tg2[l][None], tB2[l][None], wnext)
    return h


# speedup vs baseline: 1.0877x; 1.0877x over previous
import functools

import jax
import jax.numpy as jnp
from jax import lax
from jax.experimental import pallas as pl
from jax.experimental.pallas import tpu as pltpu
from jax.experimental.pallas import tpu_sc as plsc

_N = 10000
_E = 320000
_D = 128
_L = 3
_EPS = 1e-5

_NC, _NS = 2, 16
_NW = _NC * _NS
_EPW = _E // _NW
_CH = 128
_NFULL = _EPW // _CH
_TAIL = _EPW - _NFULL * _CH
_NPT = 632
_NPAD = _NS * _NPT

_ET = 1280
_NT = 1000


def _ln(x, g, b):
    m = jnp.mean(x, axis=-1, keepdims=True)
    v = jnp.mean(jnp.square(x - m), axis=-1, keepdims=True)
    return (x - m) * lax.rsqrt(v + _EPS) * g + b


def _pack_bf16(x):
    b = jax.lax.bitcast_convert_type(x.astype(jnp.bfloat16),
                                     jnp.uint16).astype(jnp.int32)
    return b[:, :64] | (b[:, 64:] << 16)


def _unpack_bf16(w):
    lo = jax.lax.bitcast_convert_type(w << 16, jnp.float32)
    hi = jax.lax.bitcast_convert_type(w & jnp.int32(-65536), jnp.float32)
    return jnp.concatenate([lo, hi], axis=1)


def _mm_body(x_ref, w_ref, o_ref):
    o_ref[...] = jnp.dot(x_ref[...], w_ref[...],
                         preferred_element_type=jnp.float32)


def _rows_matmul(x, w):
    n, k = x.shape
    return pl.pallas_call(
        _mm_body,
        grid=(n // _NT,),
        in_specs=[pl.BlockSpec((_NT, k), lambda i: (i, 0)),
                  pl.BlockSpec(w.shape, lambda i: (0, 0))],
        out_specs=pl.BlockSpec((_NT, w.shape[1]), lambda i: (i, 0)),
        out_shape=jax.ShapeDtypeStruct((n, w.shape[1]), jnp.float32),
    )(x, w)


def _edge_body(g_ref, bx_ref, w1_ref, b1_ref, g1_ref, B1_ref,
               w2_ref, b2_ref, g2_ref, B2_ref, o_ref):
    u = (g_ref[...]
         + jnp.dot(bx_ref[...], w1_ref[...], preferred_element_type=jnp.float32)
         + b1_ref[...])
    u = jnp.maximum(_ln(u, g1_ref[...], B1_ref[...]), 0.0)
    t = jnp.dot(u, w2_ref[...], preferred_element_type=jnp.float32) + b2_ref[...]
    o_ref[...] = jnp.maximum(_ln(t, g2_ref[...], B2_ref[...]), 0.0)


def _edge_mlp(gath, bond, w1b, b1, g1, B1, w2, b2, g2, B2):
    row = pl.BlockSpec((_ET, _D), lambda i: (i, 0))
    rowp = pl.BlockSpec((_ET, _D // 2), lambda i: (i, 0))

    def full(a):
        return pl.BlockSpec(a.shape, lambda i: (0,) * a.ndim)

    return pl.pallas_call(
        _edge_body,
        grid=(_E // _ET,),
        in_specs=[row, row, full(w1b), full(b1), full(g1), full(B1),
                  full(w2), full(b2), full(g2), full(B2)],
        out_specs=row,
        out_shape=jax.ShapeDtypeStruct((_E, _D), jnp.float32),
    )(gath, bond, w1b, b1, g1, B1, w2, b2, g2, B2)


def _node_body(ax_ref, p0_ref, p1_ref, w1a_ref, w1b_ref, b1_ref, g1_ref,
               B1_ref, w2_ref, b2_ref, g2_ref, B2_ref, wn_ref,
               h_ref, hp_ref):
    agg = p0_ref[...] + p1_ref[...]
    t = (jnp.dot(ax_ref[...], w1a_ref[...], preferred_element_type=jnp.float32)
         + jnp.dot(agg, w1b_ref[...], preferred_element_type=jnp.float32)
         + b1_ref[...])
    t = jnp.maximum(_ln(t, g1_ref[...], B1_ref[...]), 0.0)
    t2 = jnp.dot(t, w2_ref[...], preferred_element_type=jnp.float32) + b2_ref[...]
    h = jnp.maximum(_ln(t2, g2_ref[...], B2_ref[...]), 0.0)
    h_ref[...] = h
    hp_ref[...] = jnp.dot(h, wn_ref[...], preferred_element_type=jnp.float32)


def _node_mlp(atom, part0, part1, w1a, w1b, b1, g1, B1, w2, b2, g2, B2,
              wnext):
    grid = _N // _NT
    row = pl.BlockSpec((_NT, _D), lambda i: (i, 0))
    rowp = pl.BlockSpec((_NT, _D // 2), lambda i: (i, 0))

    def full(a):
        return pl.BlockSpec(a.shape, lambda i: (0,) * a.ndim)

    return pl.pallas_call(
        _node_body,
        grid=(grid,),
        in_specs=[row, row, row, full(w1a), full(w1b), full(b1), full(g1),
                  full(B1), full(w2), full(b2), full(g2), full(B2),
                  full(wnext)],
        out_specs=(row, row),
        out_shape=(jax.ShapeDtypeStruct((_N, _D), jnp.float32),
                   jax.ShapeDtypeStruct((_N, _D), jnp.float32)),
    )(atom, part0, part1, w1a, w1b, b1, g1, B1, w2, b2, g2, B2, wnext)


def _sc_gather(hp, src):
    mesh = plsc.VectorSubcoreMesh(core_axis_name="c", subcore_axis_name="s")

    @functools.partial(
        pl.kernel, mesh=mesh,
        out_type=jax.ShapeDtypeStruct((_E, _D), jnp.float32),
        scratch_types=[pltpu.VMEM((_CH,), jnp.int32),
                       pltpu.VMEM((_CH, _D), jnp.float32),
                       pltpu.VMEM((_TAIL,), jnp.int32),
                       pltpu.VMEM((_TAIL, _D), jnp.float32),
                       pltpu.SemaphoreType.DMA],
    )
    def gather_k(hp_hbm, src_hbm, out_hbm, idx_v, rows_v, idxt_v, rowst_v, sem):
        w = lax.axis_index("c") * _NS + lax.axis_index("s")
        base = w * _EPW

        def body(i, carry):
            off = base + i * _CH
            pltpu.sync_copy(src_hbm.at[pl.ds(off, _CH)], idx_v)
            pltpu.async_copy(hp_hbm.at[idx_v], rows_v, sem).wait()
            pltpu.sync_copy(rows_v, out_hbm.at[pl.ds(off, _CH)])
            return carry

        lax.fori_loop(0, _NFULL, body, 0)
        off = base + _NFULL * _CH
        pltpu.sync_copy(src_hbm.at[pl.ds(off, _TAIL)], idxt_v)
        pltpu.async_copy(hp_hbm.at[idxt_v], rowst_v, sem).wait()
        pltpu.sync_copy(rowst_v, out_hbm.at[pl.ds(off, _TAIL)])

    return gather_k(hp, src)


def _sc_scatter(bh, dst, zeros_tile):
    mesh = plsc.VectorSubcoreMesh(core_axis_name="c", subcore_axis_name="s")

    @functools.partial(
        pl.kernel, mesh=mesh,
        out_type=(jax.ShapeDtypeStruct((_NPAD, _D), jnp.float32),
                  jax.ShapeDtypeStruct((_NPAD, _D), jnp.float32)),
        scratch_types=[pltpu.VMEM_SHARED((_NPAD, _D), jnp.float32),
                       pltpu.VMEM((_CH,), jnp.int32),
                       pltpu.VMEM((_CH, _D), jnp.float32),
                       pltpu.VMEM((_TAIL,), jnp.int32),
                       pltpu.VMEM((_TAIL, _D), jnp.float32),
                       pltpu.SemaphoreType.DMA],
    )
    def scatter_k(bh_hbm, dst_hbm, z_hbm, out0_hbm, out1_hbm, acc, idx_v,
                  rows_v, idxt_v, rowst_v, sem):
        c = lax.axis_index("c")
        s = lax.axis_index("s")
        pltpu.sync_copy(z_hbm, acc.at[pl.ds(s * _NPT, _NPT)])
        plsc.subcore_barrier()
        base = (c * _NS + s) * _EPW

        def body(i, carry):
            off = base + i * _CH
            pltpu.sync_copy(dst_hbm.at[pl.ds(off, _CH)], idx_v)
            pltpu.sync_copy(bh_hbm.at[pl.ds(off, _CH)], rows_v)
            pltpu.sync_copy(rows_v, acc.at[idx_v], add=True)
            return carry

        lax.fori_loop(0, _NFULL, body, 0)
        off = base + _NFULL * _CH
        pltpu.sync_copy(dst_hbm.at[pl.ds(off, _TAIL)], idxt_v)
        pltpu.sync_copy(bh_hbm.at[pl.ds(off, _TAIL)], rowst_v)
        pltpu.sync_copy(rowst_v, acc.at[idxt_v], add=True)
        plsc.subcore_barrier()

        @pl.when(c == 0)
        def _():
            pltpu.sync_copy(acc.at[pl.ds(s * _NPT, _NPT)],
                            out0_hbm.at[pl.ds(s * _NPT, _NPT)])

        @pl.when(c == 1)
        def _():
            pltpu.sync_copy(acc.at[pl.ds(s * _NPT, _NPT)],
                            out1_hbm.at[pl.ds(s * _NPT, _NPT)])

    return scatter_k(bh, dst, zeros_tile)


def kernel(atom_x, bond_x, subtree_h, bW1, bb1, bg1, bB1, bW2, bb2, bg2,
           bB2, tW1, tb1, tg1, tB1, tW2, tb2, tg2, tB2, edge_index):
    src = edge_index[0]
    dst = edge_index[1]
    zeros_tile = jnp.zeros((_NPT, _D), jnp.float32)
    bond16 = bond_x.astype(jnp.bfloat16)

    hp = _rows_matmul(subtree_h, bW1[0, :_D, :])
    h = subtree_h
    for l in range(_L):
        g = _sc_gather(hp, src)
        bh = _edge_mlp(g, bond16, bW1[l, _D:, :].astype(jnp.bfloat16),
                       bb1[l][None], bg1[l][None],
                       bB1[l][None], bW2[l], bb2[l][None], bg2[l][None],
                       bB2[l][None])
        part0, part1 = _sc_scatter(bh, dst, zeros_tile)
        wnext = bW1[(l + 1) % _L, :_D, :]
        h, hp = _node_mlp(atom_x, part0, part1, tW1[l, :_D, :], tW1[l, _D:, :],
                          tb1[l][None], tg1[l][None], tB1[l][None], tW2[l],
                          tb2[l][None], tg2[l][None], tB2[l][None], wnext)
    return h


# speedup vs baseline: 1.3719x; 1.2614x over previous
import functools

import jax
import jax.numpy as jnp
from jax import lax
from jax.experimental import pallas as pl
from jax.experimental.pallas import tpu as pltpu
from jax.experimental.pallas import tpu_sc as plsc

_N = 10000
_E = 320000
_D = 128
_L = 3
_EPS = 1e-5

_NC, _NS = 2, 16
_NW = _NC * _NS
_H = _E // 2
_EPW = _H // _NW
_CH = 128
_NFULL = _EPW // _CH
_TAIL = _EPW - _NFULL * _CH
_NPT = 632
_NPAD = _NS * _NPT

_ET = 1280
_NT = 1000


def _ln(x, g, b):
    m = jnp.mean(x, axis=-1, keepdims=True)
    v = jnp.mean(jnp.square(x - m), axis=-1, keepdims=True)
    return (x - m) * lax.rsqrt(v + _EPS) * g + b


def _pack_bf16(x):
    b = jax.lax.bitcast_convert_type(x.astype(jnp.bfloat16),
                                     jnp.uint16).astype(jnp.int32)
    return b[:, :64] | (b[:, 64:] << 16)


def _unpack_bf16(w):
    lo = jax.lax.bitcast_convert_type(w << 16, jnp.float32)
    hi = jax.lax.bitcast_convert_type(w & jnp.int32(-65536), jnp.float32)
    return jnp.concatenate([lo, hi], axis=1)


def _mm_body(x_ref, w_ref, o_ref):
    o_ref[...] = jnp.dot(x_ref[...], w_ref[...],
                         preferred_element_type=jnp.float32)


def _rows_matmul(x, w):
    n, k = x.shape
    return pl.pallas_call(
        _mm_body,
        grid=(n // _NT,),
        in_specs=[pl.BlockSpec((_NT, k), lambda i: (i, 0)),
                  pl.BlockSpec(w.shape, lambda i: (0, 0))],
        out_specs=pl.BlockSpec((_NT, w.shape[1]), lambda i: (i, 0)),
        out_shape=jax.ShapeDtypeStruct((n, w.shape[1]), jnp.float32),
    )(x, w)


def _edge_body(g_ref, bx_ref, w1_ref, b1_ref, g1_ref, B1_ref,
               w2_ref, b2_ref, g2_ref, B2_ref, o_ref):
    u = (g_ref[...]
         + jnp.dot(bx_ref[...], w1_ref[...], preferred_element_type=jnp.float32)
         + b1_ref[...])
    u = jnp.maximum(_ln(u, g1_ref[...], B1_ref[...]), 0.0)
    t = jnp.dot(u, w2_ref[...], preferred_element_type=jnp.float32) + b2_ref[...]
    o_ref[...] = jnp.maximum(_ln(t, g2_ref[...], B2_ref[...]), 0.0)


def _edge_mlp(gath, bond, w1b, b1, g1, B1, w2, b2, g2, B2):
    row = pl.BlockSpec((_ET, _D), lambda i: (i, 0))
    rowp = pl.BlockSpec((_ET, _D // 2), lambda i: (i, 0))

    def full(a):
        return pl.BlockSpec(a.shape, lambda i: (0,) * a.ndim)

    return pl.pallas_call(
        _edge_body,
        grid=(_H // _ET,),
        in_specs=[row, row, full(w1b), full(b1), full(g1), full(B1),
                  full(w2), full(b2), full(g2), full(B2)],
        out_specs=row,
        out_shape=jax.ShapeDtypeStruct((_H, _D), jnp.float32),
    )(gath, bond, w1b, b1, g1, B1, w2, b2, g2, B2)


def _node_body(ax_ref, p0_ref, p1_ref, w1a_ref, w1b_ref, b1_ref, g1_ref,
               B1_ref, w2_ref, b2_ref, g2_ref, B2_ref, wn_ref,
               h_ref, hp_ref):
    agg = p0_ref[...] + p1_ref[...]
    t = (jnp.dot(ax_ref[...], w1a_ref[...], preferred_element_type=jnp.float32)
         + jnp.dot(agg, w1b_ref[...], preferred_element_type=jnp.float32)
         + b1_ref[...])
    t = jnp.maximum(_ln(t, g1_ref[...], B1_ref[...]), 0.0)
    t2 = jnp.dot(t, w2_ref[...], preferred_element_type=jnp.float32) + b2_ref[...]
    h = jnp.maximum(_ln(t2, g2_ref[...], B2_ref[...]), 0.0)
    h_ref[...] = h
    hp_ref[...] = jnp.dot(h, wn_ref[...], preferred_element_type=jnp.float32)


def _node_mlp(atom, part0, part1, w1a, w1b, b1, g1, B1, w2, b2, g2, B2,
              wnext):
    grid = _N // _NT
    row = pl.BlockSpec((_NT, _D), lambda i: (i, 0))
    rowp = pl.BlockSpec((_NT, _D // 2), lambda i: (i, 0))

    def full(a):
        return pl.BlockSpec(a.shape, lambda i: (0,) * a.ndim)

    return pl.pallas_call(
        _node_body,
        grid=(grid,),
        in_specs=[row, row, row, full(w1a), full(w1b), full(b1), full(g1),
                  full(B1), full(w2), full(b2), full(g2), full(B2),
                  full(wnext)],
        out_specs=(row, row),
        out_shape=(jax.ShapeDtypeStruct((_N, _D), jnp.float32),
                   jax.ShapeDtypeStruct((_N, _D), jnp.float32)),
    )(atom, part0, part1, w1a, w1b, b1, g1, B1, w2, b2, g2, B2, wnext)


def _sc_gather(hp, src):
    mesh = plsc.VectorSubcoreMesh(core_axis_name="c", subcore_axis_name="s")

    @functools.partial(
        pl.kernel, mesh=mesh,
        out_type=jax.ShapeDtypeStruct((_H, _D), jnp.float32),
        scratch_types=[pltpu.VMEM((_CH,), jnp.int32),
                       pltpu.VMEM((_CH, _D), jnp.float32),
                       pltpu.VMEM((_TAIL,), jnp.int32),
                       pltpu.VMEM((_TAIL, _D), jnp.float32),
                       pltpu.SemaphoreType.DMA],
    )
    def gather_k(hp_hbm, src_hbm, out_hbm, idx_v, rows_v, idxt_v, rowst_v, sem):
        w = lax.axis_index("c") * _NS + lax.axis_index("s")
        base = w * _EPW

        def body(i, carry):
            off = base + i * _CH
            pltpu.sync_copy(src_hbm.at[pl.ds(off, _CH)], idx_v)
            pltpu.async_copy(hp_hbm.at[idx_v], rows_v, sem).wait()
            pltpu.sync_copy(rows_v, out_hbm.at[pl.ds(off, _CH)])
            return carry

        lax.fori_loop(0, _NFULL, body, 0)
        off = base + _NFULL * _CH
        pltpu.sync_copy(src_hbm.at[pl.ds(off, _TAIL)], idxt_v)
        pltpu.async_copy(hp_hbm.at[idxt_v], rowst_v, sem).wait()
        pltpu.sync_copy(rowst_v, out_hbm.at[pl.ds(off, _TAIL)])

    return gather_k(hp, src)


def _sc_scatter(bh, dst, init0, init1):
    mesh = plsc.VectorSubcoreMesh(core_axis_name="c", subcore_axis_name="s")

    @functools.partial(
        pl.kernel, mesh=mesh,
        out_type=(jax.ShapeDtypeStruct((_NPAD, _D), jnp.float32),
                  jax.ShapeDtypeStruct((_NPAD, _D), jnp.float32)),
        scratch_types=[pltpu.VMEM_SHARED((_NPAD, _D), jnp.float32),
                       pltpu.VMEM((_CH,), jnp.int32),
                       pltpu.VMEM((_CH, _D), jnp.float32),
                       pltpu.VMEM((_TAIL,), jnp.int32),
                       pltpu.VMEM((_TAIL, _D), jnp.float32),
                       pltpu.SemaphoreType.DMA],
    )
    def scatter_k(bh_hbm, dst_hbm, i0_hbm, i1_hbm, out0_hbm, out1_hbm, acc,
                  idx_v, rows_v, idxt_v, rowst_v, sem):
        c = lax.axis_index("c")
        s = lax.axis_index("s")

        @pl.when(c == 0)
        def _():
            pltpu.sync_copy(i0_hbm.at[pl.ds(s * _NPT, _NPT)],
                            acc.at[pl.ds(s * _NPT, _NPT)])

        @pl.when(c == 1)
        def _():
            pltpu.sync_copy(i1_hbm.at[pl.ds(s * _NPT, _NPT)],
                            acc.at[pl.ds(s * _NPT, _NPT)])

        plsc.subcore_barrier()
        base = (c * _NS + s) * _EPW

        def body(i, carry):
            off = base + i * _CH
            pltpu.sync_copy(dst_hbm.at[pl.ds(off, _CH)], idx_v)
            pltpu.sync_copy(bh_hbm.at[pl.ds(off, _CH)], rows_v)
            pltpu.sync_copy(rows_v, acc.at[idx_v], add=True)
            return carry

        lax.fori_loop(0, _NFULL, body, 0)
        off = base + _NFULL * _CH
        pltpu.sync_copy(dst_hbm.at[pl.ds(off, _TAIL)], idxt_v)
        pltpu.sync_copy(bh_hbm.at[pl.ds(off, _TAIL)], rowst_v)
        pltpu.sync_copy(rowst_v, acc.at[idxt_v], add=True)
        plsc.subcore_barrier()

        @pl.when(c == 0)
        def _():
            pltpu.sync_copy(acc.at[pl.ds(s * _NPT, _NPT)],
                            out0_hbm.at[pl.ds(s * _NPT, _NPT)])

        @pl.when(c == 1)
        def _():
            pltpu.sync_copy(acc.at[pl.ds(s * _NPT, _NPT)],
                            out1_hbm.at[pl.ds(s * _NPT, _NPT)])

    return scatter_k(bh, dst, init0, init1)


def kernel(atom_x, bond_x, subtree_h, bW1, bb1, bg1, bB1, bW2, bb2, bg2,
           bB2, tW1, tb1, tg1, tB1, tW2, tb2, tg2, tB2, edge_index):
    src = edge_index[0]
    dst = edge_index[1]
    src_a, src_b = src[:_H], src[_H:]
    dst_a, dst_b = dst[:_H], dst[_H:]
    bond_a = bond_x[:_H].astype(jnp.bfloat16)
    bond_b = bond_x[_H:].astype(jnp.bfloat16)
    zeros_init = jnp.zeros((_NPAD, _D), jnp.float32)

    hp = _rows_matmul(subtree_h, bW1[0, :_D, :])
    h = subtree_h
    for l in range(_L):
        ew = (bW1[l, _D:, :].astype(jnp.bfloat16), bb1[l][None],
              bg1[l][None], bB1[l][None], bW2[l], bb2[l][None],
              bg2[l][None], bB2[l][None])
        g_a = _sc_gather(hp, src_a)
        g_b = _sc_gather(hp, src_b)
        bh_a = _edge_mlp(g_a, bond_a, *ew)
        bh_b = _edge_mlp(g_b, bond_b, *ew)
        pa0, pa1 = _sc_scatter(bh_a, dst_a, zeros_init, zeros_init)
        pb0, pb1 = _sc_scatter(bh_b, dst_b, pa0, pa1)
        wnext = bW1[(l + 1) % _L, :_D, :]
        h, hp = _node_mlp(atom_x, pb0, pb1, tW1[l, :_D, :], tW1[l, _D:, :],
                          tb1[l][None], tg1[l][None], tB1[l][None], tW2[l],
                          tb2[l][None], tg2[l][None], tB2[l][None], wnext)
    return h


# speedup vs baseline: 1.5282x; 1.1139x over previous
import functools

import jax
import jax.numpy as jnp
from jax import lax
from jax.experimental import pallas as pl
from jax.experimental.pallas import tpu as pltpu
from jax.experimental.pallas import tpu_sc as plsc

_N = 10000
_E = 320000
_D = 128
_L = 3
_EPS = 1e-5

_NC, _NS = 2, 16
_NW = _NC * _NS
_H = _E // 2
_EPW = _H // _NW
_CH = 128
_NFULL = _EPW // _CH
_TAIL = _EPW - _NFULL * _CH
_KG = 3
_NRG = _NFULL // _KG
_KS = 1
_NRS = _NFULL
_NPT = 632
_NPAD = _NS * _NPT

_ET = 1280
_NT = 1000


def _ln(x, g, b):
    m = jnp.mean(x, axis=-1, keepdims=True)
    v = jnp.mean(jnp.square(x - m), axis=-1, keepdims=True)
    return (x - m) * lax.rsqrt(v + _EPS) * g + b


def _pack_bf16(x):
    b = jax.lax.bitcast_convert_type(x.astype(jnp.bfloat16),
                                     jnp.uint16).astype(jnp.int32)
    return b[:, :64] | (b[:, 64:] << 16)


def _unpack_bf16(w):
    lo = jax.lax.bitcast_convert_type(w << 16, jnp.float32)
    hi = jax.lax.bitcast_convert_type(w & jnp.int32(-65536), jnp.float32)
    return jnp.concatenate([lo, hi], axis=1)


def _mm_body(x_ref, w_ref, o_ref):
    o_ref[...] = jnp.dot(x_ref[...], w_ref[...],
                         preferred_element_type=jnp.float32)


def _rows_matmul(x, w):
    n, k = x.shape
    return pl.pallas_call(
        _mm_body,
        grid=(n // _NT,),
        in_specs=[pl.BlockSpec((_NT, k), lambda i: (i, 0)),
                  pl.BlockSpec(w.shape, lambda i: (0, 0))],
        out_specs=pl.BlockSpec((_NT, w.shape[1]), lambda i: (i, 0)),
        out_shape=jax.ShapeDtypeStruct((n, w.shape[1]), jnp.float32),
    )(x, w)


def _edge_body(g_ref, bx_ref, w1_ref, b1_ref, g1_ref, B1_ref,
               w2_ref, b2_ref, g2_ref, B2_ref, o_ref):
    u = (g_ref[...]
         + jnp.dot(bx_ref[...], w1_ref[...], preferred_element_type=jnp.float32)
         + b1_ref[...])
    u = jnp.maximum(_ln(u, g1_ref[...], B1_ref[...]), 0.0)
    t = jnp.dot(u, w2_ref[...], preferred_element_type=jnp.float32) + b2_ref[...]
    o_ref[...] = jnp.maximum(_ln(t, g2_ref[...], B2_ref[...]), 0.0)


def _edge_mlp(gath, bond, w1b, b1, g1, B1, w2, b2, g2, B2):
    row = pl.BlockSpec((_ET, _D), lambda i: (i, 0))
    rowp = pl.BlockSpec((_ET, _D // 2), lambda i: (i, 0))

    def full(a):
        return pl.BlockSpec(a.shape, lambda i: (0,) * a.ndim)

    return pl.pallas_call(
        _edge_body,
        grid=(_H // _ET,),
        in_specs=[row, row, full(w1b), full(b1), full(g1), full(B1),
                  full(w2), full(b2), full(g2), full(B2)],
        out_specs=row,
        out_shape=jax.ShapeDtypeStruct((_H, _D), jnp.float32),
    )(gath, bond, w1b, b1, g1, B1, w2, b2, g2, B2)


def _node_body(ax_ref, p0_ref, p1_ref, w1a_ref, w1b_ref, b1_ref, g1_ref,
               B1_ref, w2_ref, b2_ref, g2_ref, B2_ref, wn_ref,
               h_ref, hp_ref):
    agg = p0_ref[...] + p1_ref[...]
    t = (jnp.dot(ax_ref[...], w1a_ref[...], preferred_element_type=jnp.float32)
         + jnp.dot(agg, w1b_ref[...], preferred_element_type=jnp.float32)
         + b1_ref[...])
    t = jnp.maximum(_ln(t, g1_ref[...], B1_ref[...]), 0.0)
    t2 = jnp.dot(t, w2_ref[...], preferred_element_type=jnp.float32) + b2_ref[...]
    h = jnp.maximum(_ln(t2, g2_ref[...], B2_ref[...]), 0.0)
    h_ref[...] = h
    hp_ref[...] = jnp.dot(h, wn_ref[...], preferred_element_type=jnp.float32)


def _node_mlp(atom, part0, part1, w1a, w1b, b1, g1, B1, w2, b2, g2, B2,
              wnext):
    grid = _N // _NT
    row = pl.BlockSpec((_NT, _D), lambda i: (i, 0))
    rowp = pl.BlockSpec((_NT, _D // 2), lambda i: (i, 0))

    def full(a):
        return pl.BlockSpec(a.shape, lambda i: (0,) * a.ndim)

    return pl.pallas_call(
        _node_body,
        grid=(grid,),
        in_specs=[row, row, row, full(w1a), full(w1b), full(b1), full(g1),
                  full(B1), full(w2), full(b2), full(g2), full(B2),
                  full(wnext)],
        out_specs=(row, row),
        out_shape=(jax.ShapeDtypeStruct((_N, _D), jnp.float32),
                   jax.ShapeDtypeStruct((_N, _D), jnp.float32)),
    )(atom, part0, part1, w1a, w1b, b1, g1, B1, w2, b2, g2, B2, wnext)


def _sc_gather(hp, src):
    mesh = plsc.VectorSubcoreMesh(core_axis_name="c", subcore_axis_name="s")

    @functools.partial(
        pl.kernel, mesh=mesh,
        out_type=jax.ShapeDtypeStruct((_H, _D), jnp.float32),
        scratch_types=[pltpu.VMEM((_EPW,), jnp.int32),
                       pltpu.VMEM((2, _KG * _CH, _D), jnp.float32),
                       pltpu.VMEM((_TAIL, _D), jnp.float32),
                       pltpu.SemaphoreType.DMA,
                       pltpu.SemaphoreType.DMA],
    )
    def gather_k(hp_hbm, src_hbm, out_hbm, idx_all, rows_v, rowst_v,
                 semg, semo):
        w = lax.axis_index("c") * _NS + lax.axis_index("s")
        base = w * _EPW
        pltpu.sync_copy(src_hbm.at[pl.ds(base, _EPW)], idx_all)

        gs = [None, None]
        outs = [None, None]

        def start_gathers(r):
            a = r % 2
            gs[a] = [pltpu.async_copy(
                hp_hbm.at[idx_all.at[pl.ds((r * _KG + k) * _CH, _CH)]],
                rows_v.at[a, pl.ds(k * _CH, _CH)], semg)
                for k in range(_KG)]

        start_gathers(0)
        for r in range(_NRG):
            a = r % 2
            if r + 1 < _NRG:
                if outs[1 - a] is not None:
                    outs[1 - a].wait()
                start_gathers(r + 1)
            for d in gs[a]:
                d.wait()
            outs[a] = pltpu.async_copy(
                rows_v.at[a],
                out_hbm.at[pl.ds(base + r * _KG * _CH, _KG * _CH)], semo)
        for o in outs:
            if o is not None:
                o.wait()
        off = base + _NRG * _KG * _CH
        pltpu.async_copy(hp_hbm.at[idx_all.at[pl.ds(off - base, _TAIL)]],
                         rowst_v, semg).wait()
        pltpu.sync_copy(rowst_v, out_hbm.at[pl.ds(off, _TAIL)])

    return gather_k(hp, src)


def _sc_scatter(bh, dst, init0, init1):
    mesh = plsc.VectorSubcoreMesh(core_axis_name="c", subcore_axis_name="s")

    @functools.partial(
        pl.kernel, mesh=mesh,
        out_type=(jax.ShapeDtypeStruct((_NPAD, _D), jnp.float32),
                  jax.ShapeDtypeStruct((_NPAD, _D), jnp.float32)),
        scratch_types=[pltpu.VMEM_SHARED((_NPAD, _D), jnp.float32),
                       pltpu.VMEM((2, _KS, _CH), jnp.int32),
                       pltpu.VMEM((2, _KS * _CH, _D), jnp.float32),
                       pltpu.VMEM((_TAIL,), jnp.int32),
                       pltpu.VMEM((_TAIL, _D), jnp.float32),
                       pltpu.SemaphoreType.DMA,
                       pltpu.SemaphoreType.DMA],
    )
    def scatter_k(bh_hbm, dst_hbm, i0_hbm, i1_hbm, out0_hbm, out1_hbm, acc,
                  idx_v, rows_v, idxt_v, rowst_v, seml, sema):
        c = lax.axis_index("c")
        s = lax.axis_index("s")

        @pl.when(c == 0)
        def _():
            pltpu.sync_copy(i0_hbm.at[pl.ds(s * _NPT, _NPT)],
                            acc.at[pl.ds(s * _NPT, _NPT)])

        @pl.when(c == 1)
        def _():
            pltpu.sync_copy(i1_hbm.at[pl.ds(s * _NPT, _NPT)],
                            acc.at[pl.ds(s * _NPT, _NPT)])

        plsc.subcore_barrier()
        base = (c * _NS + s) * _EPW

        loads = [None, None]
        adds = [None, None]

        def start_round(r):
            a = r % 2
            loads[a] = [
                pltpu.async_copy(bh_hbm.at[pl.ds(base + r * _CH, _CH)],
                                 rows_v.at[a], seml),
                pltpu.async_copy(dst_hbm.at[pl.ds(base + r * _CH, _CH)],
                                 idx_v.at[a, 0], seml)]

        def add_round(r):
            a = r % 2
            adds[a] = [pltpu.async_copy(rows_v.at[a],
                                        acc.at[idx_v.at[a, 0]], sema,
                                        add=True)]

        start_round(0)
        for r in range(_NRS):
            a = r % 2
            if r + 1 < _NRS:
                if adds[1 - a] is not None:
                    for d in adds[1 - a]:
                        d.wait()
                    adds[1 - a] = None
                start_round(r + 1)
            for d in loads[a]:
                d.wait()
            add_round(r)
        for ds in adds:
            if ds is not None:
                for d in ds:
                    d.wait()
        off = base + _NRS * _CH
        pltpu.sync_copy(dst_hbm.at[pl.ds(off, _TAIL)], idxt_v)
        pltpu.sync_copy(bh_hbm.at[pl.ds(off, _TAIL)], rowst_v)
        pltpu.sync_copy(rowst_v, acc.at[idxt_v], add=True)
        plsc.subcore_barrier()

        @pl.when(c == 0)
        def _():
            pltpu.sync_copy(acc.at[pl.ds(s * _NPT, _NPT)],
                            out0_hbm.at[pl.ds(s * _NPT, _NPT)])

        @pl.when(c == 1)
        def _():
            pltpu.sync_copy(acc.at[pl.ds(s * _NPT, _NPT)],
                            out1_hbm.at[pl.ds(s * _NPT, _NPT)])

    return scatter_k(bh, dst, init0, init1)


def kernel(atom_x, bond_x, subtree_h, bW1, bb1, bg1, bB1, bW2, bb2, bg2,
           bB2, tW1, tb1, tg1, tB1, tW2, tb2, tg2, tB2, edge_index):
    src = edge_index[0]
    dst = edge_index[1]
    src_a, src_b = src[:_H], src[_H:]
    dst_a, dst_b = dst[:_H], dst[_H:]
    bond_a = bond_x[:_H].astype(jnp.bfloat16)
    bond_b = bond_x[_H:].astype(jnp.bfloat16)
    zeros_init = jnp.zeros((_NPAD, _D), jnp.float32)

    hp = _rows_matmul(subtree_h, bW1[0, :_D, :])
    h = subtree_h
    for l in range(_L):
        ew = (bW1[l, _D:, :].astype(jnp.bfloat16), bb1[l][None],
              bg1[l][None], bB1[l][None], bW2[l], bb2[l][None],
              bg2[l][None], bB2[l][None])
        g_a = _sc_gather(hp, src_a)
        g_b = _sc_gather(hp, src_b)
        bh_a = _edge_mlp(g_a, bond_a, *ew)
        bh_b = _edge_mlp(g_b, bond_b, *ew)
        pa0, pa1 = _sc_scatter(bh_a, dst_a, zeros_init, zeros_init)
        pb0, pb1 = _sc_scatter(bh_b, dst_b, pa0, pa1)
        wnext = bW1[(l + 1) % _L, :_D, :]
        h, hp = _node_mlp(atom_x, pb0, pb1, tW1[l, :_D, :], tW1[l, _D:, :],
                          tb1[l][None], tg1[l][None], tB1[l][None], tW2[l],
                          tb2[l][None], tg2[l][None], tB2[l][None], wnext)
    return h
